# Initial kernel scaffold; baseline (speedup 1.0000x reference)
import jax
import jax.numpy as jnp
from jax.experimental import pallas as pl


def kernel(x, weight, expert_bias):
    raise NotImplementedError("write your pallas kernel here")


# speedup vs baseline: 2.2954x; 2.2954x over previous
import jax
import jax.numpy as jnp
from jax.experimental import pallas as pl
from jax.experimental.pallas import tpu as pltpu

N_EXPERTS = 64
TOPK = 8
N_GROUPS = 8
GROUP_SIZE = N_EXPERTS // N_GROUPS
TOPK_GROUPS = 4
ROUTE_SCALE = 2.5
BT = 512


def _router_body(x_ref, w_ref, bias_ref, w_out, i_out, s_out):
    x = x_ref[...]
    w = w_ref[...]
    logits = jax.lax.dot_general(
        x, w, (((1,), (1,)), ((), ())), preferred_element_type=jnp.float32)
    s = jax.nn.sigmoid(logits)
    s_out[...] = s
    sb = s + bias_ref[...]

    bt = s.shape[0]
    gs = jnp.concatenate(
        [jnp.max(sb[:, g * GROUP_SIZE:(g + 1) * GROUP_SIZE], axis=1,
                 keepdims=True) for g in range(N_GROUPS)], axis=1)
    gidx = jax.lax.broadcasted_iota(jnp.int32, (bt, N_GROUPS), 1)
    cnt = jnp.zeros((bt, N_GROUPS), jnp.int32)
    for j in range(N_GROUPS):
        gj = gs[:, j:j + 1]
        beats = (gj > gs) | ((gj == gs) & (j < gidx))
        cnt = cnt + beats.astype(jnp.int32)
    cnt64 = jnp.concatenate(
        [jnp.broadcast_to(cnt[:, g:g + 1], (bt, GROUP_SIZE))
         for g in range(N_GROUPS)], axis=1)
    keep64 = cnt64 < TOPK_GROUPS
    neg_inf = jnp.float32(-jnp.inf)
    m = jnp.where(keep64, sb, neg_inf)

    e_iota = jax.lax.broadcasted_iota(jnp.int32, (bt, N_EXPERTS), 1)
    vals, idxs = [], []
    for _ in range(TOPK):
        mk = jnp.max(m, axis=1, keepdims=True)
        eq = m == mk
        idx = jnp.min(jnp.where(eq, e_iota, N_EXPERTS), axis=1, keepdims=True)
        sel = e_iota == idx
        wv = jnp.sum(jnp.where(sel, s, 0.0), axis=1, keepdims=True)
        vals.append(wv)
        idxs.append(idx)
        m = jnp.where(sel, neg_inf, m)
    w_out[...] = jnp.concatenate(vals, axis=1) * ROUTE_SCALE
    i_out[...] = jnp.concatenate(idxs, axis=1)


def kernel(x, weight, expert_bias):
    B, D = x.shape
    bias = expert_bias.reshape(1, N_EXPERTS)
    grid = (B // BT,)
    weights, indices, raw = pl.pallas_call(
        _router_body,
        grid=grid,
        in_specs=[
            pl.BlockSpec((BT, D), lambda i: (i, 0)),
            pl.BlockSpec((N_EXPERTS, D), lambda i: (0, 0)),
            pl.BlockSpec((1, N_EXPERTS), lambda i: (0, 0)),
        ],
        out_specs=[
            pl.BlockSpec((BT, TOPK), lambda i: (i, 0)),
            pl.BlockSpec((BT, TOPK), lambda i: (i, 0)),
            pl.BlockSpec((BT, N_EXPERTS), lambda i: (i, 0)),
        ],
        out_shape=[
            jax.ShapeDtypeStruct((B, TOPK), jnp.float32),
            jax.ShapeDtypeStruct((B, TOPK), jnp.int32),
            jax.ShapeDtypeStruct((B, N_EXPERTS), jnp.float32),
        ],
    )(x, weight, bias)
    return weights, indices, raw


# speedup vs baseline: 4.5640x; 1.9883x over previous
import jax
import jax.numpy as jnp
from jax.experimental import pallas as pl
from jax.experimental.pallas import tpu as pltpu

N_EXPERTS = 64
TOPK = 8
N_GROUPS = 8
GROUP_SIZE = N_EXPERTS // N_GROUPS
TOPK_GROUPS = 4
ROUTE_SCALE = 2.5
BT = 512


def _router_body(x_ref, w_ref, bias_ref, w_out, i_out, s_out):
    x = x_ref[...]
    w = w_ref[...]
    logits = jax.lax.dot_general(
        x, w, (((1,), (1,)), ((), ())), preferred_element_type=jnp.float32)
    s = jax.nn.sigmoid(logits)
    s_out[...] = s

    bt = s.shape[0]
    sT = s.T
    sbT = sT + bias_ref[...]

    gs = jnp.max(sbT.reshape(N_GROUPS, GROUP_SIZE, bt), axis=1)
    gidx = jax.lax.broadcasted_iota(jnp.int32, (N_GROUPS, bt), 0)
    cnt = jnp.zeros((N_GROUPS, bt), jnp.int32)
    for j in range(N_GROUPS):
        gj = gs[j:j + 1, :]
        beats = (gj > gs) | ((gj == gs) & (j < gidx))
        cnt = cnt + beats.astype(jnp.int32)
    keep64 = jnp.broadcast_to(
        cnt[:, None, :] < TOPK_GROUPS, (N_GROUPS, GROUP_SIZE, bt)
    ).reshape(N_EXPERTS, bt)
    neg_inf = jnp.float32(-jnp.inf)
    m = jnp.where(keep64, sbT, neg_inf)

    e_iota = jax.lax.broadcasted_iota(jnp.int32, (N_EXPERTS, bt), 0)
    vals, idxs = [], []
    for _ in range(TOPK):
        mk = jnp.max(m, axis=0, keepdims=True)
        eq = m == mk
        idx = jnp.min(jnp.where(eq, e_iota, N_EXPERTS), axis=0, keepdims=True)
        sel = e_iota == idx
        wv = jnp.sum(jnp.where(sel, sT, 0.0), axis=0, keepdims=True)
        vals.append(wv)
        idxs.append(idx)
        m = jnp.where(sel, neg_inf, m)
    w_out[...] = (jnp.concatenate(vals, axis=0) * ROUTE_SCALE).T
    i_out[...] = jnp.concatenate(idxs, axis=0).T


def kernel(x, weight, expert_bias):
    B, D = x.shape
    bias = expert_bias.reshape(N_EXPERTS, 1)
    grid = (B // BT,)
    weights, indices, raw = pl.pallas_call(
        _router_body,
        grid=grid,
        in_specs=[
            pl.BlockSpec((BT, D), lambda i: (i, 0)),
            pl.BlockSpec((N_EXPERTS, D), lambda i: (0, 0)),
            pl.BlockSpec((N_EXPERTS, 1), lambda i: (0, 0)),
        ],
        out_specs=[
            pl.BlockSpec((BT, TOPK), lambda i: (i, 0)),
            pl.BlockSpec((BT, TOPK), lambda i: (i, 0)),
            pl.BlockSpec((BT, N_EXPERTS), lambda i: (i, 0)),
        ],
        out_shape=[
            jax.ShapeDtypeStruct((B, TOPK), jnp.float32),
            jax.ShapeDtypeStruct((B, TOPK), jnp.int32),
            jax.ShapeDtypeStruct((B, N_EXPERTS), jnp.float32),
        ],
    )(x, weight, bias)
    return weights, indices, raw
